# baseline (speedup 1.0000x reference)
import functools

import jax
import jax.numpy as jnp
from jax import lax
from jax.experimental import pallas as pl
from jax.experimental.pallas import tpu as pltpu
from jax.experimental.pallas import tpu_sc as plsc

N_NODES = 10000
N_EDGES = 160000
D = 256
DH = 128
NS = 16
ROWS_A = 624
ROWS_LAST = N_NODES - (NS - 1) * ROWS_A
ROW0_LAST = (NS - 1) * ROWS_A
EDGES_PER_TILE = N_EDGES // NS
AGG_CHUNK = 80
AGG_SLOTS = 4
AGG_AHEAD = 2
AGG_BLK = 2000
AGG_BLK_N = EDGES_PER_TILE // AGG_BLK
_BLK_CHUNKS = [(i * AGG_CHUNK, AGG_CHUNK) for i in range(25)]
_ALL_CHUNKS = [(b, off, sz) for b in range(AGG_BLK_N)
               for (off, sz) in _BLK_CHUNKS]
DEG_CHUNK = 2000
BN = 1000

_mesh = plsc.VectorSubcoreMesh(core_axis_name="c", subcore_axis_name="s")
_sc_params = pltpu.CompilerParams(use_tc_tiling_on_sc=False)


@functools.partial(
    pl.kernel,
    out_type=(
        jax.ShapeDtypeStruct((N_NODES, 16), jnp.float32),
        jax.ShapeDtypeStruct((N_NODES, 16), jnp.float32),
    ),
    mesh=_mesh,
    compiler_params=_sc_params,
    scratch_types=[
        pltpu.VMEM((DEG_CHUNK,), jnp.int32),
        pltpu.VMEM((DEG_CHUNK, 16), jnp.float32),
        pltpu.VMEM_SHARED((N_NODES, 16), jnp.float32),
    ],
)
def _degree_kernel(src_hbm, dst_hbm, ones_hbm, zeros_hbm,
                   deg_out_hbm, deg_in_hbm, idx_v, ones_v, shared_deg):
    c = lax.axis_index("c")
    s = lax.axis_index("s")
    row0 = pl.multiple_of(s * ROWS_A, 8)

    @pl.when(s < NS - 1)
    def _():
        pltpu.sync_copy(zeros_hbm.at[pl.ds(0, ROWS_A)],
                        shared_deg.at[pl.ds(row0, ROWS_A)])

    @pl.when(s == NS - 1)
    def _():
        pltpu.sync_copy(zeros_hbm,
                        shared_deg.at[pl.ds(ROW0_LAST, ROWS_LAST)])

    pltpu.sync_copy(ones_hbm, ones_v)
    plsc.subcore_barrier()

    def scatter_ones(ids_hbm):
        def body(j, carry):
            base = pl.multiple_of(s * EDGES_PER_TILE + j * DEG_CHUNK, 16)
            pltpu.sync_copy(ids_hbm.at[pl.ds(base, DEG_CHUNK)], idx_v)
            pltpu.sync_copy(ones_v, shared_deg.at[idx_v], add=True)
            return carry
        lax.fori_loop(0, EDGES_PER_TILE // DEG_CHUNK, body, 0)

    @pl.when(c == 0)
    def _():
        scatter_ones(src_hbm)

    @pl.when(c == 1)
    def _():
        scatter_ones(dst_hbm)

    plsc.subcore_barrier()

    def writeback(out_hbm):
        @pl.when(s < NS - 1)
        def _():
            pltpu.sync_copy(shared_deg.at[pl.ds(row0, ROWS_A)],
                            out_hbm.at[pl.ds(row0, ROWS_A)])

        @pl.when(s == NS - 1)
        def _():
            pltpu.sync_copy(shared_deg.at[pl.ds(ROW0_LAST, ROWS_LAST)],
                            out_hbm.at[pl.ds(ROW0_LAST, ROWS_LAST)])

    @pl.when(c == 0)
    def _():
        writeback(deg_out_hbm)

    @pl.when(c == 1)
    def _():
        writeback(deg_in_hbm)


@functools.partial(
    pl.kernel,
    out_type=(
        jax.ShapeDtypeStruct((N_NODES, DH), jnp.float32),
        jax.ShapeDtypeStruct((N_NODES, DH), jnp.float32),
    ),
    mesh=_mesh,
    compiler_params=_sc_params,
    scratch_types=(
        [pltpu.VMEM((AGG_BLK,), jnp.int32)] * 4
        + [pltpu.VMEM((AGG_CHUNK, DH), jnp.float32)] * AGG_SLOTS
        + [pltpu.VMEM_SHARED((N_NODES, DH), jnp.float32)]
        + [pltpu.SemaphoreType.DMA] * (2 * AGG_SLOTS + 4)
    ),
)
def _agg_kernel(h0_hbm, h1_hbm, src_hbm, dst_hbm, zeros_hbm,
                agg0_hbm, agg1_hbm, *scratch):
    stage_s = scratch[0:2]
    stage_d = scratch[2:4]
    rows = scratch[4:4 + AGG_SLOTS]
    shared_agg = scratch[4 + AGG_SLOTS]
    gsem = scratch[-2 * AGG_SLOTS - 4:-AGG_SLOTS - 4]
    ssem = scratch[-AGG_SLOTS - 4:-4]
    stgs = scratch[-4:-2]
    stgd = scratch[-2:]
    c = lax.axis_index("c")
    s = lax.axis_index("s")
    row0 = pl.multiple_of(s * ROWS_A, 8)

    @pl.when(s < NS - 1)
    def _():
        pltpu.sync_copy(zeros_hbm.at[pl.ds(0, ROWS_A)],
                        shared_agg.at[pl.ds(row0, ROWS_A)])

    @pl.when(s == NS - 1)
    def _():
        pltpu.sync_copy(zeros_hbm,
                        shared_agg.at[pl.ds(ROW0_LAST, ROWS_LAST)])

    plsc.subcore_barrier()

    def run(h_hbm):
        def stage_args(b):
            base = pl.multiple_of(s * EDGES_PER_TILE + b * AGG_BLK, 16)
            p = b % 2
            return (
                (src_hbm.at[pl.ds(base, AGG_BLK)], stage_s[p], stgs[p]),
                (dst_hbm.at[pl.ds(base, AGG_BLK)], stage_d[p], stgd[p]),
            )

        def gather_args(t):
            b, off, sz = _ALL_CHUNKS[t]
            m = t % AGG_SLOTS
            isrc = stage_s[b % 2].at[pl.ds(off, sz)]
            buf = rows[m] if sz == AGG_CHUNK else rows[m].at[pl.ds(0, sz)]
            return h_hbm.at[isrc], buf, gsem[m]

        def scatter_args(t):
            b, off, sz = _ALL_CHUNKS[t]
            m = t % AGG_SLOTS
            idx = stage_d[b % 2].at[pl.ds(off, sz)]
            buf = rows[m] if sz == AGG_CHUNK else rows[m].at[pl.ds(0, sz)]
            return buf, shared_agg.at[idx], ssem[m]

        def load_and_gather(t):
            b, off, _ = _ALL_CHUNKS[t]
            if off == 0 and b > 0:
                for a in stage_args(b):
                    pltpu.make_async_copy(*a).wait()
            if off == AGG_SLOTS * AGG_CHUNK and b + 1 < AGG_BLK_N:
                for a in stage_args(b + 1):
                    pltpu.async_copy(*a)
            pltpu.async_copy(*gather_args(t))

        for a in stage_args(0):
            pltpu.async_copy(*a)
        for a in stage_args(0):
            pltpu.make_async_copy(*a).wait()
        n_chunks = len(_ALL_CHUNKS)
        for t in range(AGG_AHEAD):
            load_and_gather(t)
        for t in range(n_chunks):
            if t + AGG_AHEAD < n_chunks:
                if t + AGG_AHEAD >= AGG_SLOTS:
                    pltpu.make_async_copy(
                        *scatter_args(t + AGG_AHEAD - AGG_SLOTS)).wait()
                load_and_gather(t + AGG_AHEAD)
            pltpu.make_async_copy(*gather_args(t)).wait()
            pltpu.async_copy(*scatter_args(t), add=True)
        for t in range(n_chunks - AGG_SLOTS, n_chunks):
            pltpu.make_async_copy(*scatter_args(t)).wait()

    @pl.when(c == 0)
    def _():
        run(h0_hbm)

    @pl.when(c == 1)
    def _():
        run(h1_hbm)

    plsc.subcore_barrier()

    def writeback(out_hbm):
        @pl.when(s < NS - 1)
        def _():
            pltpu.sync_copy(shared_agg.at[pl.ds(row0, ROWS_A)],
                            out_hbm.at[pl.ds(row0, ROWS_A)])

        @pl.when(s == NS - 1)
        def _():
            pltpu.sync_copy(shared_agg.at[pl.ds(ROW0_LAST, ROWS_LAST)],
                            out_hbm.at[pl.ds(ROW0_LAST, ROWS_LAST)])

    @pl.when(c == 0)
    def _():
        writeback(agg0_hbm)

    @pl.when(c == 1)
    def _():
        writeback(agg1_hbm)


def _scale_split_body(x_ref, deg_ref, o0_ref, o1_ref):
    ns = lax.rsqrt(jnp.maximum(deg_ref[:, 0:1], 1.0))
    xs = x_ref[...] * ns
    o0_ref[...] = xs[:, :DH]
    o1_ref[...] = xs[:, DH:]


def _scale_split(x, deg_out):
    return pl.pallas_call(
        _scale_split_body,
        grid=(N_NODES // BN,),
        in_specs=[
            pl.BlockSpec((BN, D), lambda i: (i, 0)),
            pl.BlockSpec((BN, 16), lambda i: (i, 0)),
        ],
        out_specs=[pl.BlockSpec((BN, DH), lambda i: (i, 0))] * 2,
        out_shape=[jax.ShapeDtypeStruct((N_NODES, DH), jnp.float32)] * 2,
    )(x, deg_out)


def _mid_layer_body(a0_ref, a1_ref, din_ref, dout_ref, W_ref, b_ref,
                    o0_ref, o1_ref):
    nd = lax.rsqrt(jnp.maximum(din_ref[:, 0:1], 1.0))
    h = jnp.concatenate([a0_ref[...], a1_ref[...]], axis=1) * nd
    y = jnp.dot(h, W_ref[...], preferred_element_type=jnp.float32) + b_ref[...]
    y = jnp.maximum(y, 0.0)
    ns = lax.rsqrt(jnp.maximum(dout_ref[:, 0:1], 1.0))
    y = y * ns
    o0_ref[...] = y[:, :DH]
    o1_ref[...] = y[:, DH:]


def _mid_layer(agg0, agg1, deg_in, deg_out, W, b):
    return pl.pallas_call(
        _mid_layer_body,
        grid=(N_NODES // BN,),
        in_specs=[
            pl.BlockSpec((BN, DH), lambda i: (i, 0)),
            pl.BlockSpec((BN, DH), lambda i: (i, 0)),
            pl.BlockSpec((BN, 16), lambda i: (i, 0)),
            pl.BlockSpec((BN, 16), lambda i: (i, 0)),
            pl.BlockSpec((D, D), lambda i: (0, 0)),
            pl.BlockSpec((1, D), lambda i: (0, 0)),
        ],
        out_specs=[pl.BlockSpec((BN, DH), lambda i: (i, 0))] * 2,
        out_shape=[jax.ShapeDtypeStruct((N_NODES, DH), jnp.float32)] * 2,
    )(agg0, agg1, deg_in, deg_out, W, b)


def _final_body(a0_ref, a1_ref, din_ref, W2_ref, b2_ref,
                Wf1_ref, bf1_ref, Wf2_ref, bf2_ref, ans_ref, hg_ref):
    i = pl.program_id(0)
    nd = lax.rsqrt(jnp.maximum(din_ref[:, 0:1], 1.0))
    h = jnp.concatenate([a0_ref[...], a1_ref[...]], axis=1) * nd
    y = jnp.dot(h, W2_ref[...], preferred_element_type=jnp.float32) + b2_ref[...]
    m = jnp.max(y, axis=0, keepdims=True)

    @pl.when(i == 0)
    def _():
        hg_ref[...] = m

    @pl.when(i > 0)
    def _():
        hg_ref[...] = jnp.maximum(hg_ref[...], m)

    @pl.when(i == N_NODES // BN - 1)
    def _():
        hg = hg_ref[...]
        z = jnp.dot(hg, Wf1_ref[...], preferred_element_type=jnp.float32)
        z = jnp.maximum(z + bf1_ref[...], 0.0)
        logit = jnp.dot(z, Wf2_ref[...], preferred_element_type=jnp.float32)
        logit = logit + bf2_ref[...]
        e = jnp.exp(logit - jnp.max(logit, axis=1, keepdims=True))
        ans_ref[...] = e / jnp.sum(e, axis=1, keepdims=True)


def _final(agg0, agg1, deg_in, W2, b2, Wf1, bf1, Wf2, bf2):
    return pl.pallas_call(
        _final_body,
        grid=(N_NODES // BN,),
        in_specs=[
            pl.BlockSpec((BN, DH), lambda i: (i, 0)),
            pl.BlockSpec((BN, DH), lambda i: (i, 0)),
            pl.BlockSpec((BN, 16), lambda i: (i, 0)),
            pl.BlockSpec((D, D), lambda i: (0, 0)),
            pl.BlockSpec((1, D), lambda i: (0, 0)),
            pl.BlockSpec((D, DH), lambda i: (0, 0)),
            pl.BlockSpec((1, DH), lambda i: (0, 0)),
            pl.BlockSpec((DH, 10), lambda i: (0, 0)),
            pl.BlockSpec((1, 10), lambda i: (0, 0)),
        ],
        out_specs=[
            pl.BlockSpec((1, 10), lambda i: (0, 0)),
            pl.BlockSpec((1, D), lambda i: (0, 0)),
        ],
        out_shape=[
            jax.ShapeDtypeStruct((1, 10), jnp.float32),
            jax.ShapeDtypeStruct((1, D), jnp.float32),
        ],
    )(agg0, agg1, deg_in, W2, b2, Wf1, bf1, Wf2, bf2)


def kernel(x, edge_index, W1, b1, W2, b2, Wf1, bf1, Wf2, bf2):
    src = edge_index[0].astype(jnp.int32)
    dst = edge_index[1].astype(jnp.int32)
    ones16 = jnp.ones((DEG_CHUNK, 16), jnp.float32)
    zeros16 = jnp.zeros((ROWS_LAST, 16), jnp.float32)
    zeros128 = jnp.zeros((ROWS_LAST, DH), jnp.float32)

    deg_out, deg_in = _degree_kernel(src, dst, ones16, zeros16)
    xs0, xs1 = _scale_split(x, deg_out)
    agg0, agg1 = _agg_kernel(xs0, xs1, src, dst, zeros128)
    h0, h1 = _mid_layer(agg0, agg1, deg_in, deg_out, W1, b1.reshape(1, D))
    agg0b, agg1b = _agg_kernel(h0, h1, src, dst, zeros128)
    ans, hg = _final(agg0b, agg1b, deg_in, W2, b2.reshape(1, D),
                     Wf1, bf1.reshape(1, DH), Wf2, bf2.reshape(1, 10))
    return (ans, hg)


# speedup vs baseline: 1.0588x; 1.0588x over previous
import functools

import jax
import jax.numpy as jnp
from jax import lax
from jax.experimental import pallas as pl
from jax.experimental.pallas import tpu as pltpu
from jax.experimental.pallas import tpu_sc as plsc

N_NODES = 10000
N_EDGES = 160000
D = 256
DH = 128
NS = 16
ROWS_A = 624
ROWS_LAST = N_NODES - (NS - 1) * ROWS_A
ROW0_LAST = (NS - 1) * ROWS_A
EDGES_PER_TILE = N_EDGES // NS
AGG_CHUNK = 112
AGG_SLOTS = 3
AGG_AHEAD = 2
AGG_BLK = 2000
AGG_BLK_N = EDGES_PER_TILE // AGG_BLK
_BLK_CHUNKS = [(i * AGG_CHUNK, AGG_CHUNK) for i in range(17)] + [(1904, 96)]
_ALL_CHUNKS = [(b, off, sz) for b in range(AGG_BLK_N)
               for (off, sz) in _BLK_CHUNKS]
DEG_CHUNK = 2000
BN = 1000

_mesh = plsc.VectorSubcoreMesh(core_axis_name="c", subcore_axis_name="s")
_sc_params = pltpu.CompilerParams(use_tc_tiling_on_sc=False)


@functools.partial(
    pl.kernel,
    out_type=(
        jax.ShapeDtypeStruct((N_NODES, 16), jnp.float32),
        jax.ShapeDtypeStruct((N_NODES, 16), jnp.float32),
    ),
    mesh=_mesh,
    compiler_params=_sc_params,
    scratch_types=[
        pltpu.VMEM((DEG_CHUNK,), jnp.int32),
        pltpu.VMEM((DEG_CHUNK, 16), jnp.float32),
        pltpu.VMEM_SHARED((N_NODES, 16), jnp.float32),
    ],
)
def _degree_kernel(src_hbm, dst_hbm, ones_hbm, zeros_hbm,
                   deg_out_hbm, deg_in_hbm, idx_v, ones_v, shared_deg):
    c = lax.axis_index("c")
    s = lax.axis_index("s")
    row0 = pl.multiple_of(s * ROWS_A, 8)

    @pl.when(s < NS - 1)
    def _():
        pltpu.sync_copy(zeros_hbm.at[pl.ds(0, ROWS_A)],
                        shared_deg.at[pl.ds(row0, ROWS_A)])

    @pl.when(s == NS - 1)
    def _():
        pltpu.sync_copy(zeros_hbm,
                        shared_deg.at[pl.ds(ROW0_LAST, ROWS_LAST)])

    pltpu.sync_copy(ones_hbm, ones_v)
    plsc.subcore_barrier()

    def scatter_ones(ids_hbm):
        def body(j, carry):
            base = pl.multiple_of(s * EDGES_PER_TILE + j * DEG_CHUNK, 16)
            pltpu.sync_copy(ids_hbm.at[pl.ds(base, DEG_CHUNK)], idx_v)
            pltpu.sync_copy(ones_v, shared_deg.at[idx_v], add=True)
            return carry
        lax.fori_loop(0, EDGES_PER_TILE // DEG_CHUNK, body, 0)

    @pl.when(c == 0)
    def _():
        scatter_ones(src_hbm)

    @pl.when(c == 1)
    def _():
        scatter_ones(dst_hbm)

    plsc.subcore_barrier()

    def writeback(out_hbm):
        @pl.when(s < NS - 1)
        def _():
            pltpu.sync_copy(shared_deg.at[pl.ds(row0, ROWS_A)],
                            out_hbm.at[pl.ds(row0, ROWS_A)])

        @pl.when(s == NS - 1)
        def _():
            pltpu.sync_copy(shared_deg.at[pl.ds(ROW0_LAST, ROWS_LAST)],
                            out_hbm.at[pl.ds(ROW0_LAST, ROWS_LAST)])

    @pl.when(c == 0)
    def _():
        writeback(deg_out_hbm)

    @pl.when(c == 1)
    def _():
        writeback(deg_in_hbm)


@functools.partial(
    pl.kernel,
    out_type=(
        jax.ShapeDtypeStruct((N_NODES, DH), jnp.float32),
        jax.ShapeDtypeStruct((N_NODES, DH), jnp.float32),
    ),
    mesh=_mesh,
    compiler_params=_sc_params,
    scratch_types=(
        [pltpu.VMEM((AGG_BLK,), jnp.int32)] * 4
        + [pltpu.VMEM((AGG_CHUNK, DH), jnp.float32)] * AGG_SLOTS
        + [pltpu.VMEM_SHARED((N_NODES, DH), jnp.float32)]
        + [pltpu.SemaphoreType.DMA] * (2 * AGG_SLOTS + 4)
    ),
)
def _agg_kernel(h0_hbm, h1_hbm, src_hbm, dst_hbm, zeros_hbm,
                agg0_hbm, agg1_hbm, *scratch):
    stage_s = scratch[0:2]
    stage_d = scratch[2:4]
    rows = scratch[4:4 + AGG_SLOTS]
    shared_agg = scratch[4 + AGG_SLOTS]
    gsem = scratch[-2 * AGG_SLOTS - 4:-AGG_SLOTS - 4]
    ssem = scratch[-AGG_SLOTS - 4:-4]
    stgs = scratch[-4:-2]
    stgd = scratch[-2:]
    c = lax.axis_index("c")
    s = lax.axis_index("s")
    row0 = pl.multiple_of(s * ROWS_A, 8)

    @pl.when(s < NS - 1)
    def _():
        pltpu.sync_copy(zeros_hbm.at[pl.ds(0, ROWS_A)],
                        shared_agg.at[pl.ds(row0, ROWS_A)])

    @pl.when(s == NS - 1)
    def _():
        pltpu.sync_copy(zeros_hbm,
                        shared_agg.at[pl.ds(ROW0_LAST, ROWS_LAST)])

    plsc.subcore_barrier()

    def run(h_hbm):
        def stage_args(b):
            base = pl.multiple_of(s * EDGES_PER_TILE + b * AGG_BLK, 16)
            p = b % 2
            return (
                (src_hbm.at[pl.ds(base, AGG_BLK)], stage_s[p], stgs[p]),
                (dst_hbm.at[pl.ds(base, AGG_BLK)], stage_d[p], stgd[p]),
            )

        def gather_args(t):
            b, off, sz = _ALL_CHUNKS[t]
            m = t % AGG_SLOTS
            isrc = stage_s[b % 2].at[pl.ds(off, sz)]
            buf = rows[m] if sz == AGG_CHUNK else rows[m].at[pl.ds(0, sz)]
            return h_hbm.at[isrc], buf, gsem[m]

        def scatter_args(t):
            b, off, sz = _ALL_CHUNKS[t]
            m = t % AGG_SLOTS
            idx = stage_d[b % 2].at[pl.ds(off, sz)]
            buf = rows[m] if sz == AGG_CHUNK else rows[m].at[pl.ds(0, sz)]
            return buf, shared_agg.at[idx], ssem[m]

        def load_and_gather(t):
            b, off, _ = _ALL_CHUNKS[t]
            if off == 0 and b > 0:
                for a in stage_args(b):
                    pltpu.make_async_copy(*a).wait()
            if off == AGG_SLOTS * AGG_CHUNK and b + 1 < AGG_BLK_N:
                for a in stage_args(b + 1):
                    pltpu.async_copy(*a)
            pltpu.async_copy(*gather_args(t))

        for a in stage_args(0):
            pltpu.async_copy(*a)
        for a in stage_args(0):
            pltpu.make_async_copy(*a).wait()
        n_chunks = len(_ALL_CHUNKS)
        for t in range(AGG_AHEAD):
            load_and_gather(t)
        for t in range(n_chunks):
            if t + AGG_AHEAD < n_chunks:
                if t + AGG_AHEAD >= AGG_SLOTS:
                    pltpu.make_async_copy(
                        *scatter_args(t + AGG_AHEAD - AGG_SLOTS)).wait()
                load_and_gather(t + AGG_AHEAD)
            pltpu.make_async_copy(*gather_args(t)).wait()
            pltpu.async_copy(*scatter_args(t), add=True)
        for t in range(n_chunks - AGG_SLOTS, n_chunks):
            pltpu.make_async_copy(*scatter_args(t)).wait()

    @pl.when(c == 0)
    def _():
        run(h0_hbm)

    @pl.when(c == 1)
    def _():
        run(h1_hbm)

    plsc.subcore_barrier()

    def writeback(out_hbm):
        @pl.when(s < NS - 1)
        def _():
            pltpu.sync_copy(shared_agg.at[pl.ds(row0, ROWS_A)],
                            out_hbm.at[pl.ds(row0, ROWS_A)])

        @pl.when(s == NS - 1)
        def _():
            pltpu.sync_copy(shared_agg.at[pl.ds(ROW0_LAST, ROWS_LAST)],
                            out_hbm.at[pl.ds(ROW0_LAST, ROWS_LAST)])

    @pl.when(c == 0)
    def _():
        writeback(agg0_hbm)

    @pl.when(c == 1)
    def _():
        writeback(agg1_hbm)


def _scale_split_body(x_ref, deg_ref, o0_ref, o1_ref):
    ns = lax.rsqrt(jnp.maximum(deg_ref[:, 0:1], 1.0))
    xs = x_ref[...] * ns
    o0_ref[...] = xs[:, :DH]
    o1_ref[...] = xs[:, DH:]


def _scale_split(x, deg_out):
    return pl.pallas_call(
        _scale_split_body,
        grid=(N_NODES // BN,),
        in_specs=[
            pl.BlockSpec((BN, D), lambda i: (i, 0)),
            pl.BlockSpec((BN, 16), lambda i: (i, 0)),
        ],
        out_specs=[pl.BlockSpec((BN, DH), lambda i: (i, 0))] * 2,
        out_shape=[jax.ShapeDtypeStruct((N_NODES, DH), jnp.float32)] * 2,
    )(x, deg_out)


def _mid_layer_body(a0_ref, a1_ref, din_ref, dout_ref, W_ref, b_ref,
                    o0_ref, o1_ref):
    nd = lax.rsqrt(jnp.maximum(din_ref[:, 0:1], 1.0))
    h = jnp.concatenate([a0_ref[...], a1_ref[...]], axis=1) * nd
    y = jnp.dot(h, W_ref[...], preferred_element_type=jnp.float32) + b_ref[...]
    y = jnp.maximum(y, 0.0)
    ns = lax.rsqrt(jnp.maximum(dout_ref[:, 0:1], 1.0))
    y = y * ns
    o0_ref[...] = y[:, :DH]
    o1_ref[...] = y[:, DH:]


def _mid_layer(agg0, agg1, deg_in, deg_out, W, b):
    return pl.pallas_call(
        _mid_layer_body,
        grid=(N_NODES // BN,),
        in_specs=[
            pl.BlockSpec((BN, DH), lambda i: (i, 0)),
            pl.BlockSpec((BN, DH), lambda i: (i, 0)),
            pl.BlockSpec((BN, 16), lambda i: (i, 0)),
            pl.BlockSpec((BN, 16), lambda i: (i, 0)),
            pl.BlockSpec((D, D), lambda i: (0, 0)),
            pl.BlockSpec((1, D), lambda i: (0, 0)),
        ],
        out_specs=[pl.BlockSpec((BN, DH), lambda i: (i, 0))] * 2,
        out_shape=[jax.ShapeDtypeStruct((N_NODES, DH), jnp.float32)] * 2,
    )(agg0, agg1, deg_in, deg_out, W, b)


def _final_body(a0_ref, a1_ref, din_ref, W2_ref, b2_ref,
                Wf1_ref, bf1_ref, Wf2_ref, bf2_ref, ans_ref, hg_ref):
    i = pl.program_id(0)
    nd = lax.rsqrt(jnp.maximum(din_ref[:, 0:1], 1.0))
    h = jnp.concatenate([a0_ref[...], a1_ref[...]], axis=1) * nd
    y = jnp.dot(h, W2_ref[...], preferred_element_type=jnp.float32) + b2_ref[...]
    m = jnp.max(y, axis=0, keepdims=True)

    @pl.when(i == 0)
    def _():
        hg_ref[...] = m

    @pl.when(i > 0)
    def _():
        hg_ref[...] = jnp.maximum(hg_ref[...], m)

    @pl.when(i == N_NODES // BN - 1)
    def _():
        hg = hg_ref[...]
        z = jnp.dot(hg, Wf1_ref[...], preferred_element_type=jnp.float32)
        z = jnp.maximum(z + bf1_ref[...], 0.0)
        logit = jnp.dot(z, Wf2_ref[...], preferred_element_type=jnp.float32)
        logit = logit + bf2_ref[...]
        e = jnp.exp(logit - jnp.max(logit, axis=1, keepdims=True))
        ans_ref[...] = e / jnp.sum(e, axis=1, keepdims=True)


def _final(agg0, agg1, deg_in, W2, b2, Wf1, bf1, Wf2, bf2):
    return pl.pallas_call(
        _final_body,
        grid=(N_NODES // BN,),
        in_specs=[
            pl.BlockSpec((BN, DH), lambda i: (i, 0)),
            pl.BlockSpec((BN, DH), lambda i: (i, 0)),
            pl.BlockSpec((BN, 16), lambda i: (i, 0)),
            pl.BlockSpec((D, D), lambda i: (0, 0)),
            pl.BlockSpec((1, D), lambda i: (0, 0)),
            pl.BlockSpec((D, DH), lambda i: (0, 0)),
            pl.BlockSpec((1, DH), lambda i: (0, 0)),
            pl.BlockSpec((DH, 10), lambda i: (0, 0)),
            pl.BlockSpec((1, 10), lambda i: (0, 0)),
        ],
        out_specs=[
            pl.BlockSpec((1, 10), lambda i: (0, 0)),
            pl.BlockSpec((1, D), lambda i: (0, 0)),
        ],
        out_shape=[
            jax.ShapeDtypeStruct((1, 10), jnp.float32),
            jax.ShapeDtypeStruct((1, D), jnp.float32),
        ],
    )(agg0, agg1, deg_in, W2, b2, Wf1, bf1, Wf2, bf2)


def kernel(x, edge_index, W1, b1, W2, b2, Wf1, bf1, Wf2, bf2):
    src = edge_index[0].astype(jnp.int32)
    dst = edge_index[1].astype(jnp.int32)
    ones16 = jnp.ones((DEG_CHUNK, 16), jnp.float32)
    zeros16 = jnp.zeros((ROWS_LAST, 16), jnp.float32)
    zeros128 = jnp.zeros((ROWS_LAST, DH), jnp.float32)

    deg_out, deg_in = _degree_kernel(src, dst, ones16, zeros16)
    xs0, xs1 = _scale_split(x, deg_out)
    agg0, agg1 = _agg_kernel(xs0, xs1, src, dst, zeros128)
    h0, h1 = _mid_layer(agg0, agg1, deg_in, deg_out, W1, b1.reshape(1, D))
    agg0b, agg1b = _agg_kernel(h0, h1, src, dst, zeros128)
    ans, hg = _final(agg0b, agg1b, deg_in, W2, b2.reshape(1, D),
                     Wf1, bf1.reshape(1, DH), Wf2, bf2.reshape(1, 10))
    return (ans, hg)
